# baseline (speedup 1.0000x reference)
import functools

import jax
import jax.numpy as jnp
from jax import lax
from jax.experimental import pallas as pl
from jax.experimental.pallas import tpu as pltpu

_B = 8192
_D = 1024
_P = 64
_K = 8
_BLK = 512
_EPS = 1e-12


def _body(x_ref, k_ref, v_ref, n2x_ref, n2k_ref, out_ref, loss_ref, idx_ref):
    i = pl.program_id(0)
    nprog = pl.num_programs(0)

    x = x_ref[...]
    xn = x / jnp.maximum(jnp.sqrt(n2x_ref[...]), _EPS)
    k = k_ref[...]
    kn = k / jnp.maximum(jnp.sqrt(n2k_ref[...]), _EPS)

    s = lax.dot_general(xn, kn, (((1,), (1,)), ((), ())),
                        preferred_element_type=jnp.float32)
    m = jnp.max(s, axis=1, keepdims=True)
    e = jnp.exp(s - m)
    p = e / jnp.sum(e, axis=1, keepdims=True)

    cols = lax.broadcasted_iota(jnp.int32, (_BLK, _P), 1)
    v = v_ref[...]
    v_hi = v.astype(jnp.bfloat16)
    v_lo = (v - v_hi.astype(jnp.float32)).astype(jnp.bfloat16)
    vcat = jnp.concatenate([v_hi, v_lo], axis=0)
    cols2 = lax.broadcasted_iota(jnp.int32, (_BLK, 2 * _P), 1) & (_P - 1)

    work = p
    val_sum = jnp.zeros((), jnp.float32)
    for j in range(_K):
        mx = jnp.max(work, axis=1, keepdims=True)
        amx = jnp.min(jnp.where(work == mx, cols, _P), axis=1,
                      keepdims=True)
        oh2 = jnp.where(cols2 == amx, 1.0, 0.0).astype(jnp.bfloat16)
        sel = lax.dot_general(oh2, vcat, (((1,), (0,)), ((), ())),
                              preferred_element_type=jnp.float32)
        out_ref[:, j * _D:(j + 1) * _D] = sel
        idx_ref[:, j] = amx[:, 0]
        val_sum = val_sum + jnp.sum(mx)
        work = jnp.where(cols == amx, -1.0, work)

    @pl.when(i == 0)
    def _():
        loss_ref[0, 0] = 0.0

    loss_ref[0, 0] += val_sum

    @pl.when(i == nprog - 1)
    def _():
        loss_ref[0, 0] = loss_ref[0, 0] * (-1.0 / _B)


@functools.partial(jax.jit, static_argnames=())
def _run(input_data, prompt_keys, prompt_values):
    grid = _B // _BLK
    n2x = jnp.sum(jnp.abs(input_data) ** 2, axis=-1, keepdims=True)
    n2k = jnp.sum(jnp.abs(prompt_keys) ** 2, axis=-1, keepdims=True)
    sel, loss, idxs = pl.pallas_call(
        _body,
        grid=(grid,),
        in_specs=[
            pl.BlockSpec((_BLK, _D), lambda i: (i, 0)),
            pl.BlockSpec((_P, _D), lambda i: (0, 0)),
            pl.BlockSpec((_P, _D), lambda i: (0, 0)),
            pl.BlockSpec((_BLK, 1), lambda i: (i, 0)),
            pl.BlockSpec((_P, 1), lambda i: (0, 0)),
        ],
        out_specs=[
            pl.BlockSpec((_BLK, _K * _D), lambda i: (i, 0)),
            pl.BlockSpec((1, 1), lambda i: (0, 0),
                         memory_space=pltpu.SMEM),
            pl.BlockSpec((_BLK, _K), lambda i: (i, 0)),
        ],
        out_shape=[
            jax.ShapeDtypeStruct((_B, _K * _D), jnp.float32),
            jax.ShapeDtypeStruct((1, 1), jnp.float32),
            jax.ShapeDtypeStruct((_B, _K), jnp.int32),
        ],
        compiler_params=pltpu.CompilerParams(
            dimension_semantics=("arbitrary",),
        ),
    )(input_data, prompt_keys, prompt_values, n2x, n2k)
    return sel.reshape(_B, _K, _D), loss[0, 0], idxs


def kernel(input_data, prompt_keys, prompt_values, top_k):
    del top_k
    return _run(input_data, prompt_keys, prompt_values)


# speedup vs baseline: 2.4292x; 2.4292x over previous
import functools

import jax
import jax.numpy as jnp
from jax import lax
from jax.experimental import pallas as pl
from jax.experimental.pallas import tpu as pltpu

_B = 8192
_D = 1024
_P = 64
_K = 8
_BLK = 512
_EPS = 1e-12


def _body(x_ref, k_ref, v_ref, n2x_ref, n2k_ref, out_ref, loss_ref, idx_ref):
    i = pl.program_id(0)
    nprog = pl.num_programs(0)

    x = x_ref[...]
    xn = x / jnp.maximum(jnp.sqrt(n2x_ref[...]), _EPS)
    k = k_ref[...]
    kn = k / jnp.maximum(jnp.sqrt(n2k_ref[...]), _EPS)

    s = lax.dot_general(xn, kn, (((1,), (1,)), ((), ())),
                        preferred_element_type=jnp.float32)
    m = jnp.max(s, axis=1, keepdims=True)
    e = jnp.exp(s - m)
    p = e / jnp.sum(e, axis=1, keepdims=True)

    cols = lax.broadcasted_iota(jnp.int32, (_BLK, _P), 1)
    v = v_ref[...]
    v_hi = v.astype(jnp.bfloat16)
    v_lo = (v - v_hi.astype(jnp.float32)).astype(jnp.bfloat16)
    vcat = jnp.concatenate([v_hi, v_lo], axis=0)

    work = p
    val_sum = jnp.zeros((), jnp.float32)
    rank = jnp.full((_BLK, _P), 127, jnp.int32)
    for j in range(_K):
        mx = jnp.max(work, axis=1, keepdims=True)
        amx = jnp.min(jnp.where(work == mx, cols, _P), axis=1,
                      keepdims=True)
        rank = jnp.where(cols == amx, j, rank)
        idx_ref[:, j] = amx[:, 0]
        val_sum = val_sum + jnp.sum(mx)
        work = jnp.where(cols == amx, -1.0, work)

    rank2 = jnp.concatenate([rank, rank], axis=1)
    rank_rep = jnp.broadcast_to(rank2[:, None, :],
                                (_BLK, _K, 2 * _P)).reshape(_BLK * _K, 2 * _P)
    jmod = lax.broadcasted_iota(jnp.int32, (_BLK * _K, 2 * _P), 0) & (_K - 1)
    ohm = jnp.where(rank_rep == jmod, 1.0, 0.0).astype(jnp.bfloat16)
    out_ref[...] = lax.dot_general(ohm, vcat, (((1,), (0,)), ((), ())),
                                   preferred_element_type=jnp.float32)

    @pl.when(i == 0)
    def _():
        loss_ref[0, 0] = 0.0

    loss_ref[0, 0] += val_sum

    @pl.when(i == nprog - 1)
    def _():
        loss_ref[0, 0] = loss_ref[0, 0] * (-1.0 / _B)


@functools.partial(jax.jit, static_argnames=())
def _run(input_data, prompt_keys, prompt_values):
    grid = _B // _BLK
    n2x = jnp.sum(jnp.abs(input_data) ** 2, axis=-1, keepdims=True)
    n2k = jnp.sum(jnp.abs(prompt_keys) ** 2, axis=-1, keepdims=True)
    sel, loss, idxs = pl.pallas_call(
        _body,
        grid=(grid,),
        in_specs=[
            pl.BlockSpec((_BLK, _D), lambda i: (i, 0)),
            pl.BlockSpec((_P, _D), lambda i: (0, 0)),
            pl.BlockSpec((_P, _D), lambda i: (0, 0)),
            pl.BlockSpec((_BLK, 1), lambda i: (i, 0)),
            pl.BlockSpec((_P, 1), lambda i: (0, 0)),
        ],
        out_specs=[
            pl.BlockSpec((_BLK * _K, _D), lambda i: (i, 0)),
            pl.BlockSpec((1, 1), lambda i: (0, 0),
                         memory_space=pltpu.SMEM),
            pl.BlockSpec((_BLK, _K), lambda i: (i, 0)),
        ],
        out_shape=[
            jax.ShapeDtypeStruct((_B * _K, _D), jnp.float32),
            jax.ShapeDtypeStruct((1, 1), jnp.float32),
            jax.ShapeDtypeStruct((_B, _K), jnp.int32),
        ],
        compiler_params=pltpu.CompilerParams(
            dimension_semantics=("arbitrary",),
        ),
    )(input_data, prompt_keys, prompt_values, n2x, n2k)
    return sel.reshape(_B, _K, _D), loss[0, 0], idxs


def kernel(input_data, prompt_keys, prompt_values, top_k):
    del top_k
    return _run(input_data, prompt_keys, prompt_values)
